# baseline (device time: 15924 ns/iter reference)
import jax
import jax.numpy as jnp
from jax import lax
from jax.experimental import pallas as pl
from jax.experimental.pallas import tpu as pltpu

N_DEV = 4
B = 2
SQ = 256
SKV = 256
HQ = 4
DH = 64
D = 512
HD = HQ * DH
HALO = 128
WIN = 128
KW = SKV + 2 * HALO


def _body(x_ref, wq_ref, k_ref, v_ref, wo_ref, out_ref,
          k_bf, v_bf, kleft, kright, vleft, vright,
          send_sems, recv_sems):
    bf16 = jnp.bfloat16
    pos = lax.axis_index("i")
    left = (pos - 1) % N_DEV
    right = (pos + 1) % N_DEV

    barrier_sem = pltpu.get_barrier_semaphore()
    for nbr in (left, right):
        pl.semaphore_signal(
            barrier_sem, inc=1,
            device_id=(nbr,), device_id_type=pl.DeviceIdType.MESH,
        )
    pl.semaphore_wait(barrier_sem, 2)

    for b in range(B):
        for half in range(2):
            rows = slice(half * HALO, (half + 1) * HALO)
            k_bf[2 * b + half] = (
                k_ref[b, rows].reshape(HALO, HD).astype(bf16))
            v_bf[2 * b + half] = (
                v_ref[b, rows].reshape(HALO, HD).astype(bf16))

    transfers = []
    for b in range(B):
        transfers += [
            (k_bf.at[2 * b + 1], kleft.at[b], right),
            (v_bf.at[2 * b + 1], vleft.at[b], right),
            (k_bf.at[2 * b], kright.at[b], left),
            (v_bf.at[2 * b], vright.at[b], left),
        ]
    rdmas = []
    for idx, (src, dst, tgt) in enumerate(transfers):
        rdma = pltpu.make_async_remote_copy(
            src_ref=src, dst_ref=dst,
            send_sem=send_sems.at[idx], recv_sem=recv_sems.at[idx],
            device_id=(tgt,), device_id_type=pl.DeviceIdType.MESH,
        )
        rdma.start()
        rdmas.append(rdma)

    wq = wq_ref[...].astype(bf16)
    q = []
    for b in range(B):
        qb = lax.dot_general(
            x_ref[b].astype(bf16), wq,
            (((1,), (0,)), ((), ())), preferred_element_type=jnp.float32,
        )
        q.append((qb * 0.125).astype(bf16))
    wo = wo_ref[...].astype(bf16)

    i2 = lax.broadcasted_iota(jnp.int32, (SQ, KW), 0)
    j2 = lax.broadcasted_iota(jnp.int32, (SQ, KW), 1)
    ki = pos * SKV - HALO + j2
    mask = (jnp.abs(i2 + HALO - j2) <= WIN) & (ki >= 0) & (ki < N_DEV * SKV)
    bias = jnp.where(mask, 0.0, -1e9).astype(jnp.float32)

    for rdma in rdmas:
        rdma.wait()

    for b in range(B):
        kfull = jnp.concatenate(
            [kleft[b], k_bf[2 * b], k_bf[2 * b + 1], kright[b]], axis=0)
        vfull = jnp.concatenate(
            [vleft[b], v_bf[2 * b], v_bf[2 * b + 1], vright[b]], axis=0)
        ctx_heads = []
        for h in range(HQ):
            cols = slice(h * DH, (h + 1) * DH)
            s = lax.dot_general(
                q[b][:, cols], kfull[:, cols], (((1,), (1,)), ((), ())),
                preferred_element_type=jnp.float32,
            ) + bias
            e = jnp.exp(s)
            w = e * (1.0 / jnp.sum(e, axis=1, keepdims=True))
            ctx = lax.dot_general(
                w.astype(bf16), vfull[:, cols],
                (((1,), (0,)), ((), ())), preferred_element_type=jnp.float32,
            )
            ctx_heads.append(ctx)
        ctx_b = jnp.concatenate(ctx_heads, axis=1).astype(bf16)
        out_ref[b] = lax.dot_general(
            ctx_b, wo,
            (((1,), (0,)), ((), ())), preferred_element_type=jnp.float32,
        )


def kernel(x, Wq, K_ext, V_ext, Wo):
    halo = pltpu.VMEM((B, HALO, HD), jnp.bfloat16)
    return pl.pallas_call(
        _body,
        out_shape=jax.ShapeDtypeStruct((B, SQ, D), jnp.float32),
        in_specs=[pl.BlockSpec(memory_space=pltpu.VMEM)] * 5,
        out_specs=pl.BlockSpec(memory_space=pltpu.VMEM),
        scratch_shapes=[
            pltpu.VMEM((B * 2, HALO, HD), jnp.bfloat16),
            pltpu.VMEM((B * 2, HALO, HD), jnp.bfloat16),
            halo, halo, halo, halo,
            pltpu.SemaphoreType.DMA((8,)),
            pltpu.SemaphoreType.DMA((8,)),
        ],
        compiler_params=pltpu.CompilerParams(collective_id=0),
    )(x, Wq, K_ext, V_ext, Wo)
